# baseline (speedup 1.0000x reference)
import functools

import jax
import jax.numpy as jnp
from jax import lax
from jax.experimental import pallas as pl
from jax.experimental.pallas import tpu as pltpu
from jax.experimental.pallas import tpu_sc as plsc

NC = 2
NS = 16
NW = NC * NS

CH = 128


_BLKV = 8064
_NREG = 62
_H = _BLKV * _NREG


def _tr(t, eye):
    return lax.dot_general(t, eye, (((0,), (0,)), ((), ())),
                           preferred_element_type=jnp.float32)


def _repack_body(t0_ref, t1_ref, eye_ref, o_ref):
    i = pl.program_id(0)
    D = t0_ref.shape[0]
    eye = eye_ref[...]

    @pl.when(i < _NREG)
    def _():
        o_ref[:, :D] = _tr(t0_ref[...], eye)
        o_ref[:, D:] = _tr(t1_ref[...], eye)

    @pl.when(i == _NREG)
    def _():
        t = t0_ref[...]
        o_ref[:32, :D] = _tr(t[:, :32], eye)
        o_ref[:32, D:] = _tr(t[:, 32:64], eye)


def _repack(tT):
    D, V = tT.shape
    eye = jnp.eye(D, dtype=jnp.float32)
    return pl.pallas_call(
        _repack_body,
        grid=(_NREG + 1,),
        in_specs=[
            pl.BlockSpec((D, _BLKV),
                         lambda i: (0, jnp.where(i == _NREG, 2 * _NREG, i))),
            pl.BlockSpec((D, _BLKV),
                         lambda i: (0, jnp.where(i == _NREG, 0, i + _NREG))),
            pl.BlockSpec((D, D), lambda i: (0, 0)),
        ],
        out_specs=pl.BlockSpec((_BLKV, 2 * D), lambda i: (i, 0)),
        out_shape=jax.ShapeDtypeStruct((V // 2, 2 * D), jnp.float32),
    )(tT, tT, eye)


def _make_gather(B, D):
    b_per_w = B // NW
    nch = b_per_w // CH
    mesh = plsc.VectorSubcoreMesh(core_axis_name="c", subcore_axis_name="s")

    @functools.partial(
        pl.kernel,
        mesh=mesh,
        compiler_params=pltpu.CompilerParams(use_tc_tiling_on_sc=False),
        out_type=jax.ShapeDtypeStruct((B, D), jnp.float32),
        scratch_types=[
            pltpu.VMEM((nch, CH), jnp.int32),
            pltpu.VMEM((CH, D), jnp.float32),
            pltpu.VMEM((CH, D), jnp.float32),
            pltpu.SemaphoreType.DMA,
            pltpu.SemaphoreType.DMA,
        ],
    )
    def gather_k(idx_hbm, table_hbm, out_hbm, idx_v, rows0, rows1, s0, s1):
        wid = lax.axis_index("s") * NC + lax.axis_index("c")
        base = wid * b_per_w
        pltpu.sync_copy(idx_hbm.at[wid], idx_v)
        pltpu.async_copy(table_hbm.at[idx_v.at[0]], rows0, s0)
        pltpu.async_copy(table_hbm.at[idx_v.at[1]], rows1, s1)

        def half(j, rows, sem):
            pltpu.make_async_copy(table_hbm.at[idx_v.at[j]], rows, sem).wait()
            pltpu.sync_copy(rows, out_hbm.at[pl.ds(base + j * CH, CH)])

            @pl.when(j + 2 < nch)
            def _():
                pltpu.async_copy(table_hbm.at[idx_v.at[j + 2]], rows, sem)

        def body(j2, carry):
            half(2 * j2, rows0, s0)
            half(2 * j2 + 1, rows1, s1)
            return carry

        lax.fori_loop(0, nch // 2, body, 0)

    return gather_k


def _mm_compute(e_ref, w_ref, o_ref):
    y = lax.dot_general(
        w_ref[...], e_ref[0],
        (((1,), (1,)), ((), ())),
        preferred_element_type=jnp.float32,
    )
    half = y.shape[1]
    O = o_ref.shape[1]
    o_ref[0, :, :half] = y[:O, :]
    o_ref[0, :, half:] = y[O:, :]


def _mm_body_a(e_ref, w_ref, o_ref):
    _mm_compute(e_ref, w_ref, o_ref)


def _mm_body_b(e_ref, w_ref, prev_ref, o_ref):
    del prev_ref
    _mm_compute(e_ref, w_ref, o_ref)


def _matmul_chunked(ep_a, ep_b, w2, L, Bt, O):
    half = Bt // 2
    Lh = L // 2
    out_sh = jax.ShapeDtypeStruct((L, O, Bt), jnp.float32)
    out_a = pl.pallas_call(
        _mm_body_a,
        grid=(Lh,),
        in_specs=[
            pl.BlockSpec((1, half, w2.shape[0]), lambda i: (i, 0, 0)),
            pl.BlockSpec(w2.shape, lambda i: (0, 0)),
        ],
        out_specs=pl.BlockSpec((1, O, Bt), lambda i: (i, 0, 0)),
        out_shape=out_sh,
    )(ep_a, w2)
    return pl.pallas_call(
        _mm_body_b,
        grid=(Lh,),
        in_specs=[
            pl.BlockSpec((1, half, w2.shape[0]), lambda i: (i, 0, 0)),
            pl.BlockSpec(w2.shape, lambda i: (0, 0)),
            pl.BlockSpec(memory_space=pl.ANY),
        ],
        out_specs=pl.BlockSpec((1, O, Bt), lambda i, Lh_=Lh: (i + Lh_, 0, 0)),
        out_shape=out_sh,
        input_output_aliases={2: 0},
    )(ep_b, w2, out_a)


def kernel(x, emb_table, fc_w):
    Bt, L = x.shape
    B = Bt * L
    D = emb_table.shape[1]
    O = fc_w.shape[0]

    tT = jnp.transpose(emb_table)
    t2d = _repack(tT)
    t_rm = t2d.reshape(emb_table.shape)

    xT = jnp.transpose(x)
    xp = jnp.transpose(xT.reshape(L, 2, Bt // 2), (0, 2, 1))
    xp = jnp.where(
        xp < _H, 2 * xp,
        jnp.where(xp < 2 * _H, 2 * (xp - _H) + 1,
                  jnp.where(xp < 2 * _H + 32, 2 * xp - 2 * _H,
                            2 * xp - 999999)))
    B2 = B // 2
    idx2 = xp.reshape(2, NW, (B2 // NW) // CH, CH).astype(jnp.int32)

    g = _make_gather(B2, D)
    e_a = g(idx2[0], t_rm)
    e_b = g(idx2[1], t_rm)
    ep_a = e_a.reshape(L // 2, Bt // 2, 2 * D)
    ep_b = e_b.reshape(L // 2, Bt // 2, 2 * D)

    w2 = jnp.zeros((2 * D, 2 * O), jnp.float32)
    w2 = w2.at[:O, :D].set(fc_w).at[O:, D:].set(fc_w)
    out_t = _matmul_chunked(ep_a, ep_b, w2, L, Bt, O)
    return jnp.transpose(out_t, (2, 0, 1))


# speedup vs baseline: 1.1513x; 1.1513x over previous
import functools

import jax
import jax.numpy as jnp
from jax import lax
from jax.experimental import pallas as pl
from jax.experimental.pallas import tpu as pltpu
from jax.experimental.pallas import tpu_sc as plsc

NC = 2
NS = 16
NW = NC * NS

CH = 128
_NBUF = 4


_BLKV = 8064
_NREG = 62
_H = _BLKV * _NREG


def _tr(t, eye):
    return lax.dot_general(t, eye, (((0,), (0,)), ((), ())),
                           preferred_element_type=jnp.float32)


def _repack_body(t0_ref, t1_ref, eye_ref, o_ref):
    i = pl.program_id(0)
    D = t0_ref.shape[0]
    eye = eye_ref[...]

    @pl.when(i < _NREG)
    def _():
        o_ref[:, :D] = _tr(t0_ref[...], eye)
        o_ref[:, D:] = _tr(t1_ref[...], eye)

    @pl.when(i == _NREG)
    def _():
        t = t0_ref[...]
        o_ref[:32, :D] = _tr(t[:, :32], eye)
        o_ref[:32, D:] = _tr(t[:, 32:64], eye)


def _repack(tT):
    D, V = tT.shape
    eye = jnp.eye(D, dtype=jnp.float32)
    return pl.pallas_call(
        _repack_body,
        grid=(_NREG + 1,),
        in_specs=[
            pl.BlockSpec((D, _BLKV),
                         lambda i: (0, jnp.where(i == _NREG, 2 * _NREG, i))),
            pl.BlockSpec((D, _BLKV),
                         lambda i: (0, jnp.where(i == _NREG, 0, i + _NREG))),
            pl.BlockSpec((D, D), lambda i: (0, 0)),
        ],
        out_specs=pl.BlockSpec((_BLKV, 2 * D), lambda i: (i, 0)),
        out_shape=jax.ShapeDtypeStruct((V // 2, 2 * D), jnp.float32),
    )(tT, tT, eye)


def _make_gather(B, D):
    b_per_w = B // NW
    nch = b_per_w // CH
    mesh = plsc.VectorSubcoreMesh(core_axis_name="c", subcore_axis_name="s")

    @functools.partial(
        pl.kernel,
        mesh=mesh,
        compiler_params=pltpu.CompilerParams(use_tc_tiling_on_sc=False),
        out_type=jax.ShapeDtypeStruct((B, D), jnp.float32),
        scratch_types=[
            pltpu.VMEM((nch, CH), jnp.int32),
        ]
        + [pltpu.VMEM((CH, D), jnp.float32) for _ in range(_NBUF)]
        + [pltpu.SemaphoreType.DMA for _ in range(_NBUF)],
    )
    def gather_k(idx_hbm, table_hbm, out_hbm, idx_v, *bufs_sems):
        rows = bufs_sems[:_NBUF]
        sems = bufs_sems[_NBUF:]
        wid = lax.axis_index("s") * NC + lax.axis_index("c")
        base = wid * b_per_w
        pltpu.sync_copy(idx_hbm.at[wid], idx_v)
        for b in range(_NBUF):
            pltpu.async_copy(table_hbm.at[idx_v.at[b]], rows[b], sems[b])

        def step(j, rows_b, sem_b):
            pltpu.make_async_copy(table_hbm.at[idx_v.at[j]], rows_b,
                                  sem_b).wait()
            pltpu.sync_copy(rows_b, out_hbm.at[pl.ds(base + j * CH, CH)])

            @pl.when(j + _NBUF < nch)
            def _():
                pltpu.async_copy(table_hbm.at[idx_v.at[j + _NBUF]], rows_b,
                                 sem_b)

        def body(jg, carry):
            for b in range(_NBUF):
                step(_NBUF * jg + b, rows[b], sems[b])
            return carry

        lax.fori_loop(0, nch // _NBUF, body, 0)

    return gather_k


def _mm_compute(e_ref, w_ref, o_ref):
    y = lax.dot_general(
        w_ref[...], e_ref[0],
        (((1,), (1,)), ((), ())),
        preferred_element_type=jnp.float32,
    )
    half = y.shape[1]
    O = o_ref.shape[1]
    o_ref[0, :, :half] = y[:O, :]
    o_ref[0, :, half:] = y[O:, :]


def _matmul(ep3, w2, L, Bt, O):
    half = Bt // 2
    return pl.pallas_call(
        _mm_compute,
        grid=(L,),
        in_specs=[
            pl.BlockSpec((1, half, w2.shape[0]), lambda i: (i, 0, 0)),
            pl.BlockSpec(w2.shape, lambda i: (0, 0)),
        ],
        out_specs=pl.BlockSpec((1, O, Bt), lambda i: (i, 0, 0)),
        out_shape=jax.ShapeDtypeStruct((L, O, Bt), jnp.float32),
    )(ep3, w2)


def kernel(x, emb_table, fc_w):
    Bt, L = x.shape
    B = Bt * L
    D = emb_table.shape[1]
    O = fc_w.shape[0]

    tT = jnp.transpose(emb_table)
    t2d = _repack(tT)
    t_rm = t2d.reshape(emb_table.shape)

    xT = jnp.transpose(x)
    xp = jnp.transpose(xT.reshape(L, 2, Bt // 2), (0, 2, 1))
    xp = jnp.where(
        xp < _H, 2 * xp,
        jnp.where(xp < 2 * _H, 2 * (xp - _H) + 1,
                  jnp.where(xp < 2 * _H + 32, 2 * xp - 2 * _H,
                            2 * xp - 999999)))
    idx = xp.reshape(NW, (B // NW) // CH, CH).astype(jnp.int32)

    e = _make_gather(B, D)(idx, t_rm)
    ep3 = e.reshape(L, Bt // 2, 2 * D)

    w2 = jnp.zeros((2 * D, 2 * O), jnp.float32)
    w2 = w2.at[:O, :D].set(fc_w).at[O:, D:].set(fc_w)
    out_t = _matmul(ep3, w2, L, Bt, O)
    return jnp.transpose(out_t, (2, 0, 1))
